# baseline (speedup 1.0000x reference)
import numpy as np

import jax
import jax.numpy as jnp
from jax import lax
from jax.experimental import pallas as pl

_M = 256
_D = 256
_N = _M * _M
_AB = 8


def _scalar_stage(x_ref, xT_ref, mem_ref, memT_ref, t_ref):
    x = x_ref[...]
    xT = xT_ref[...]
    mem = mem_ref[...]
    memT = memT_ref[...]
    f32 = jnp.float32
    rx = 1.0 / jnp.maximum(jnp.sqrt(jnp.sum(x * x, axis=1, keepdims=True)), 1e-8)
    xn_row = x * rx
    xn_col = xT * rx
    q1_row = jnp.sum(memT, axis=0, keepdims=True)
    q2_row = jnp.sum(memT * memT, axis=0, keepdims=True)
    q2_col = jnp.sum(mem * mem, axis=1, keepdims=True)
    rn_row = 1.0 / jnp.maximum(jnp.sqrt(q2_row), 1e-8)
    rn_col = 1.0 / jnp.maximum(jnp.sqrt(q2_col), 1e-8)
    mx_col = jnp.dot(mem, xn_col, preferred_element_type=f32)
    mx_row = jnp.dot(xn_row, memT, preferred_element_type=f32)
    sims_col = mx_col * rn_col
    sims_row = mx_row * rn_row
    act_col = (sims_col > 0.3).astype(f32)
    act_row = (sims_row > 0.3).astype(f32)
    raw = jnp.dot(mem, memT, preferred_element_type=f32)
    s = act_col * (raw * rn_col * rn_row) + sims_row
    v = xn_row + jnp.dot(act_row * rn_row, mem,
                         preferred_element_type=f32) * (1.0 / _M)
    sv = jnp.sum(v, axis=1, keepdims=True)
    dv_row = jnp.dot(v, memT, preferred_element_type=f32)
    den = jnp.maximum(jnp.sqrt(q2_row + 2.0 * s * q1_row + float(_D) * s * s),
                      1e-8)
    t_ref[...] = s + (dv_row + s * sv) / den


_R0 = (13, 15, 26, 6)
_R1 = (17, 29, 16, 24)
_K0 = np.uint32(0)
_K1 = np.uint32(42)
_K2 = np.uint32(np.uint32(0x1BD11BDA) ^ _K0 ^ _K1)
_KS = (_K0, _K1, _K2)
_LO = np.nextafter(np.float32(-1), np.float32(0), dtype=np.float32)
_SCALE = np.float32(np.float32(1.0) - _LO)


def _rotl(v, r):
    return (v << np.uint32(r)) | (v >> np.uint32(32 - r))


def _noise(j):
    x0 = jnp.zeros_like(j) + _K0
    x1 = j + _K1
    for i, rots in enumerate((_R0, _R1, _R0, _R1, _R0)):
        for r in rots:
            x0 = x0 + x1
            x1 = _rotl(x1, r)
            x1 = x1 ^ x0
        x0 = x0 + _KS[(i + 1) % 3]
        x1 = x1 + _KS[(i + 2) % 3] + np.uint32(i + 1)
    bits = x0 ^ x1
    fb = (bits >> np.uint32(9)) | np.uint32(0x3F800000)
    f = lax.bitcast_convert_type(fb, jnp.float32) - np.float32(1.0)
    u = jnp.maximum(jnp.float32(_LO), f * _SCALE + _LO)
    w = -jnp.log(np.float32(1.0) - u * u)
    wc = w - np.float32(2.5)
    p1 = jnp.full_like(w, 2.81022636e-08)
    for c in (3.43273939e-07, -3.5233877e-06, -4.39150654e-06, 0.00021858087,
              -0.00125372503, -0.00417768164, 0.246640727, 1.50140941):
        p1 = np.float32(c) + p1 * wc
    wt = jnp.sqrt(w) - np.float32(3.0)
    p2 = jnp.full_like(w, -0.000200214257)
    for c in (0.000100950558, 0.00134934322, -0.00367342844, 0.00573950773,
              -0.0076224613, 0.00943887047, 1.00167406, 2.83297682):
        p2 = np.float32(c) + p2 * wt
    p = jnp.where(w < 5.0, p1, p2)
    return (p * u) * np.float32(0.1 * np.sqrt(2.0))


def _assemble(mem_ref, t_ref, out_ref):
    g = pl.program_id(0)
    shp = (_AB, _M, _D)
    a_io = lax.broadcasted_iota(jnp.uint32, shp, 0)
    i_io = lax.broadcasted_iota(jnp.uint32, shp, 1)
    d_io = lax.broadcasted_iota(jnp.uint32, shp, 2)
    base = (g * (_AB * _M * _D)).astype(jnp.uint32)
    j = base + (a_io << np.uint32(16)) + (i_io << np.uint32(8)) + d_io
    out_ref[...] = _noise(j) + mem_ref[...][None, :, :] + t_ref[...]


def kernel(x, mem):
    t = pl.pallas_call(
        _scalar_stage,
        out_shape=jax.ShapeDtypeStruct((_M, _M), jnp.float32),
    )(x, x.T, mem, mem.T)
    out3 = pl.pallas_call(
        _assemble,
        grid=(_M // _AB,),
        in_specs=[
            pl.BlockSpec((_M, _D), lambda i: (0, 0)),
            pl.BlockSpec((_AB, _M, 1), lambda i: (i, 0, 0)),
        ],
        out_specs=pl.BlockSpec((_AB, _M, _D), lambda i: (i, 0, 0)),
        out_shape=jax.ShapeDtypeStruct((_M, _M, _D), jnp.float32),
    )(mem, t[:, :, None])
    return out3.reshape(_N, _D)
